# Initial kernel scaffold; baseline (speedup 1.0000x reference)
import functools

import jax
import jax.numpy as jnp
from jax import lax
from jax.experimental import pallas as pl
from jax.experimental.pallas import tpu as pltpu
from jax.experimental.pallas import tpu_sc as plsc

N_NODES = 10000
N_EDGES = 320000
D_IN = 128
D_H1 = 64
D_H2 = 32

NC = 2
NS = 16
NW = NC * NS
E_PER_W = N_EDGES // NW
BLK = 80
NB = E_PER_W // BLK
ROWS_PER_SUB = N_NODES // NS

_MESH = plsc.VectorSubcoreMesh(core_axis_name="c", subcore_axis_name="s")

_F32 = jnp.float32


def _sc_segsum(y, src_r, dst_r, ones_blk, zD, z16, with_deg):
    d = y.shape[1]
    out_type = [jax.ShapeDtypeStruct((NC, N_NODES, d), _F32)]
    scratch = [
        pltpu.VMEM((NB, BLK), jnp.int32),
        pltpu.VMEM((NB, BLK), jnp.int32),
        pltpu.VMEM((BLK, d), _F32),
        pltpu.VMEM_SHARED((N_NODES, d), _F32),
        pltpu.SemaphoreType.DMA,
    ]
    if with_deg:
        out_type.append(jax.ShapeDtypeStruct((NC, N_NODES, 16), _F32))
        scratch += [
            pltpu.VMEM((BLK, 16), _F32),
            pltpu.VMEM_SHARED((N_NODES, 16), _F32),
        ]

    @functools.partial(
        pl.kernel,
        out_type=tuple(out_type),
        mesh=_MESH,
        scratch_types=tuple(scratch),
    )
    def k(y_hbm, src_hbm, dst_hbm, ones_hbm, zD_hbm, z16_hbm, *rest):
        if with_deg:
            (out_hbm, deg_hbm, src_v, dst_v, buf_v, acc_sh, sem,
             ones_v, deg_sh) = rest
        else:
            out_hbm, src_v, dst_v, buf_v, acc_sh, sem = rest
        c = lax.axis_index("c")
        s = lax.axis_index("s")
        wid = s * NC + c
        r0 = s * ROWS_PER_SUB
        pltpu.sync_copy(zD_hbm.at[pl.ds(r0, ROWS_PER_SUB)],
                        acc_sh.at[pl.ds(r0, ROWS_PER_SUB)])
        if with_deg:
            pltpu.sync_copy(z16_hbm.at[pl.ds(r0, ROWS_PER_SUB)],
                            deg_sh.at[pl.ds(r0, ROWS_PER_SUB)])
            pltpu.sync_copy(ones_hbm, ones_v)
        pltpu.sync_copy(src_hbm.at[wid], src_v)
        pltpu.sync_copy(dst_hbm.at[wid], dst_v)
        plsc.subcore_barrier()

        @pl.loop(0, NB)
        def _(b):
            pltpu.async_copy(y_hbm.at[src_v.at[b]], buf_v, sem).wait()
            pltpu.sync_copy(buf_v, acc_sh.at[dst_v.at[b]], add=True)
            if with_deg:
                pltpu.sync_copy(ones_v, deg_sh.at[dst_v.at[b]], add=True)

        plsc.subcore_barrier()
        pltpu.sync_copy(acc_sh.at[pl.ds(r0, ROWS_PER_SUB)],
                        out_hbm.at[c, pl.ds(r0, ROWS_PER_SUB)])
        if with_deg:
            pltpu.sync_copy(deg_sh.at[pl.ds(r0, ROWS_PER_SUB)],
                            deg_hbm.at[c, pl.ds(r0, ROWS_PER_SUB)])

    return k(y, src_r, dst_r, ones_blk, zD, z16)


def _dot(a, b):
    return lax.dot_general(a, b, (((1,), (0,)), ((), ())),
                           preferred_element_type=_F32,
                           precision=lax.Precision.HIGHEST)


def _tc_proj1(x, W_l, W_r, b):
    def body(x_ref, wl_ref, wr_ref, b_ref, y_ref, r_ref):
        xv = x_ref[...]
        y_ref[...] = _dot(xv, wl_ref[...])
        r_ref[...] = _dot(xv, wr_ref[...]) + b_ref[...]

    return pl.pallas_call(
        body,
        out_shape=(jax.ShapeDtypeStruct((N_NODES, D_H1), _F32),
                   jax.ShapeDtypeStruct((N_NODES, D_H1), _F32)),
    )(x, W_l, W_r, b)


def _tc_mid(p1, pdeg, r1, W2_l, W2_r, b2):
    def body(p1_ref, pdeg_ref, r1_ref, wl_ref, wr_ref, b_ref,
             y2_ref, r2_ref):
        deg = pdeg_ref[0, :, 0:1] + pdeg_ref[1, :, 0:1]
        degc = jnp.maximum(deg, 1.0)
        h = jnp.maximum((p1_ref[0] + p1_ref[1]) / degc + r1_ref[...], 0.0)
        y2_ref[...] = _dot(h, wl_ref[...])
        r2_ref[...] = _dot(h, wr_ref[...]) + b_ref[...]

    return pl.pallas_call(
        body,
        out_shape=(jax.ShapeDtypeStruct((N_NODES, D_H2), _F32),
                   jax.ShapeDtypeStruct((N_NODES, D_H2), _F32)),
    )(p1, pdeg, r1, W2_l, W2_r, b2)


def _tc_final(p2, pdeg, r2):
    def body(p2_ref, pdeg_ref, r2_ref, o_ref):
        deg = pdeg_ref[0, :, 0:1] + pdeg_ref[1, :, 0:1]
        degc = jnp.maximum(deg, 1.0)
        o_ref[...] = (p2_ref[0] + p2_ref[1]) / degc + r2_ref[...]

    return pl.pallas_call(
        body,
        out_shape=jax.ShapeDtypeStruct((N_NODES, D_H2), _F32),
    )(p2, pdeg, r2)


def kernel(x, edge_index, W1_l, W1_r, b1, W2_l, W2_r, b2):
    src_r = edge_index[0].reshape(NW, NB, BLK)
    dst_r = edge_index[1].reshape(NW, NB, BLK)
    ones_blk = jnp.ones((BLK, 16), _F32)
    z64 = jnp.zeros((N_NODES, D_H1), _F32)
    z32 = jnp.zeros((N_NODES, D_H2), _F32)
    z16 = jnp.zeros((N_NODES, 16), _F32)

    y1, r1 = _tc_proj1(x, W1_l, W1_r, b1.reshape(1, D_H1))
    p1, pdeg = _sc_segsum(y1, src_r, dst_r, ones_blk, z64, z16, True)
    y2, r2 = _tc_mid(p1, pdeg, r1, W2_l, W2_r, b2.reshape(1, D_H2))
    p2 = _sc_segsum(y2, src_r, dst_r, ones_blk, z32, z16, False)
    if isinstance(p2, (tuple, list)):
        p2 = p2[0]
    return _tc_final(p2, pdeg, r2)


# speedup vs baseline: 9.3994x; 9.3994x over previous
import functools

import jax
import jax.numpy as jnp
from jax import lax
from jax.experimental import pallas as pl
from jax.experimental.pallas import tpu as pltpu
from jax.experimental.pallas import tpu_sc as plsc

N_NODES = 10000
N_EDGES = 320000
D_IN = 128
D_H1 = 64
D_H2 = 32

NC = 2
NS = 16
NW = NC * NS
E_PER_W = N_EDGES // NW
BLK = 80
NB = E_PER_W // BLK
ROWS_PER_SUB = 632
N_PAD = ROWS_PER_SUB * NS

_MESH = plsc.VectorSubcoreMesh(core_axis_name="c", subcore_axis_name="s")

_F32 = jnp.float32


def _sc_segsum(y, src_r, dst_r, ones_blk, zD, z16, with_deg):
    d = y.shape[1]
    out_type = [jax.ShapeDtypeStruct((NC, N_PAD, d), _F32)]
    scratch = [
        pltpu.VMEM((NB, BLK), jnp.int32),
        pltpu.VMEM((NB, BLK), jnp.int32),
        pltpu.VMEM((BLK, d), _F32),
        pltpu.VMEM_SHARED((N_PAD, d), _F32),
        pltpu.SemaphoreType.DMA,
    ]
    if with_deg:
        out_type.append(jax.ShapeDtypeStruct((NC, N_PAD, 16), _F32))
        scratch += [
            pltpu.VMEM((BLK, 16), _F32),
            pltpu.VMEM_SHARED((N_PAD, 16), _F32),
        ]

    @functools.partial(
        pl.kernel,
        out_type=tuple(out_type),
        mesh=_MESH,
        scratch_types=tuple(scratch),
        compiler_params=pltpu.CompilerParams(use_tc_tiling_on_sc=False),
    )
    def k(y_hbm, src_hbm, dst_hbm, ones_hbm, zD_hbm, z16_hbm, *rest):
        if with_deg:
            (out_hbm, deg_hbm, src_v, dst_v, buf_v, acc_sh, sem,
             ones_v, deg_sh) = rest
        else:
            out_hbm, src_v, dst_v, buf_v, acc_sh, sem = rest
        c = lax.axis_index("c")
        s = lax.axis_index("s")
        wid = s * NC + c
        r0 = s * ROWS_PER_SUB
        pltpu.sync_copy(zD_hbm.at[pl.ds(r0, ROWS_PER_SUB)],
                        acc_sh.at[pl.ds(r0, ROWS_PER_SUB)])
        if with_deg:
            pltpu.sync_copy(z16_hbm.at[pl.ds(r0, ROWS_PER_SUB)],
                            deg_sh.at[pl.ds(r0, ROWS_PER_SUB)])
            pltpu.sync_copy(ones_hbm, ones_v)
        pltpu.sync_copy(src_hbm.at[wid], src_v)
        pltpu.sync_copy(dst_hbm.at[wid], dst_v)
        plsc.subcore_barrier()

        @pl.loop(0, NB)
        def _(b):
            pltpu.async_copy(y_hbm.at[src_v.at[b]], buf_v, sem).wait()
            pltpu.sync_copy(buf_v, acc_sh.at[dst_v.at[b]], add=True)
            if with_deg:
                pltpu.sync_copy(ones_v, deg_sh.at[dst_v.at[b]], add=True)

        plsc.subcore_barrier()
        pltpu.sync_copy(acc_sh.at[pl.ds(r0, ROWS_PER_SUB)],
                        out_hbm.at[c, pl.ds(r0, ROWS_PER_SUB)])
        if with_deg:
            pltpu.sync_copy(deg_sh.at[pl.ds(r0, ROWS_PER_SUB)],
                            deg_hbm.at[c, pl.ds(r0, ROWS_PER_SUB)])

    return k(y, src_r, dst_r, ones_blk, zD, z16)


def _dot(a, b):
    return lax.dot_general(a, b, (((1,), (0,)), ((), ())),
                           preferred_element_type=_F32,
                           precision=lax.Precision.HIGHEST)


def _tc_proj1(x, W_l, W_r, b):
    def body(x_ref, wl_ref, wr_ref, b_ref, y_ref, r_ref):
        xv = x_ref[...]
        y_ref[...] = _dot(xv, wl_ref[...])
        r_ref[...] = _dot(xv, wr_ref[...]) + b_ref[...]

    return pl.pallas_call(
        body,
        out_shape=(jax.ShapeDtypeStruct((N_NODES, D_H1), _F32),
                   jax.ShapeDtypeStruct((N_NODES, D_H1), _F32)),
    )(x, W_l, W_r, b)


def _tc_mid(p1, pdeg, r1, W2_l, W2_r, b2):
    def body(p1_ref, pdeg_ref, r1_ref, wl_ref, wr_ref, b_ref,
             y2_ref, r2_ref):
        deg = pdeg_ref[0, :N_NODES, 0:1] + pdeg_ref[1, :N_NODES, 0:1]
        degc = jnp.maximum(deg, 1.0)
        h = jnp.maximum(
            (p1_ref[0, :N_NODES, :] + p1_ref[1, :N_NODES, :]) / degc
            + r1_ref[...], 0.0)
        y2_ref[...] = _dot(h, wl_ref[...])
        r2_ref[...] = _dot(h, wr_ref[...]) + b_ref[...]

    return pl.pallas_call(
        body,
        out_shape=(jax.ShapeDtypeStruct((N_NODES, D_H2), _F32),
                   jax.ShapeDtypeStruct((N_NODES, D_H2), _F32)),
    )(p1, pdeg, r1, W2_l, W2_r, b2)


def _tc_final(p2, pdeg, r2):
    def body(p2_ref, pdeg_ref, r2_ref, o_ref):
        deg = pdeg_ref[0, :N_NODES, 0:1] + pdeg_ref[1, :N_NODES, 0:1]
        degc = jnp.maximum(deg, 1.0)
        o_ref[...] = ((p2_ref[0, :N_NODES, :] + p2_ref[1, :N_NODES, :])
                      / degc + r2_ref[...])

    return pl.pallas_call(
        body,
        out_shape=jax.ShapeDtypeStruct((N_NODES, D_H2), _F32),
    )(p2, pdeg, r2)


def kernel(x, edge_index, W1_l, W1_r, b1, W2_l, W2_r, b2):
    src_r = edge_index[0].reshape(NW, NB, BLK)
    dst_r = edge_index[1].reshape(NW, NB, BLK)
    ones_blk = jnp.ones((BLK, 16), _F32)
    z64 = jnp.zeros((N_PAD, D_H1), _F32)
    z32 = jnp.zeros((N_PAD, D_H2), _F32)
    z16 = jnp.zeros((N_PAD, 16), _F32)

    y1, r1 = _tc_proj1(x, W1_l, W1_r, b1.reshape(1, D_H1))
    p1, pdeg = _sc_segsum(y1, src_r, dst_r, ones_blk, z64, z16, True)
    y2, r2 = _tc_mid(p1, pdeg, r1, W2_l, W2_r, b2.reshape(1, D_H2))
    p2 = _sc_segsum(y2, src_r, dst_r, ones_blk, z32, z16, False)
    if isinstance(p2, (tuple, list)):
        p2 = p2[0]
    return _tc_final(p2, pdeg, r2)


# speedup vs baseline: 16.0408x; 1.7066x over previous
import functools

import jax
import jax.numpy as jnp
from jax import lax
from jax.experimental import pallas as pl
from jax.experimental.pallas import tpu as pltpu
from jax.experimental.pallas import tpu_sc as plsc

N_NODES = 10000
N_EDGES = 320000
D_IN = 128
D_H1 = 64
D_H2 = 32

NC = 2
NS = 16
NW = NC * NS
E_PER_W = N_EDGES // NW
BLK = 80
NB = E_PER_W // BLK
ROWS_PER_SUB = 632
N_PAD = ROWS_PER_SUB * NS
NSLOT = 5
LOOKAHEAD = 3

_MESH = plsc.VectorSubcoreMesh(core_axis_name="c", subcore_axis_name="s")

_F32 = jnp.float32


def _sc_segsum(y, src_r, dst_r, ones_blk, zD, z16, with_deg):
    d = y.shape[1]
    out_type = [jax.ShapeDtypeStruct((NC, N_PAD, d), _F32)]
    scratch = [
        pltpu.VMEM((NB, BLK), jnp.int32),
        pltpu.VMEM((NB, BLK), jnp.int32),
        pltpu.VMEM((NSLOT, BLK, d), _F32),
        pltpu.VMEM_SHARED((N_PAD, d), _F32),
        pltpu.SemaphoreType.DMA((NSLOT,)),
        pltpu.SemaphoreType.DMA((NSLOT,)),
    ]
    if with_deg:
        out_type.append(jax.ShapeDtypeStruct((NC, N_PAD, 16), _F32))
        scratch += [
            pltpu.VMEM((BLK, 16), _F32),
            pltpu.VMEM_SHARED((N_PAD, 16), _F32),
            pltpu.SemaphoreType.DMA((NSLOT,)),
        ]

    @functools.partial(
        pl.kernel,
        out_type=tuple(out_type),
        mesh=_MESH,
        scratch_types=tuple(scratch),
        compiler_params=pltpu.CompilerParams(use_tc_tiling_on_sc=False),
    )
    def k(y_hbm, src_hbm, dst_hbm, ones_hbm, zD_hbm, z16_hbm, *rest):
        if with_deg:
            (out_hbm, deg_hbm, src_v, dst_v, buf_v, acc_sh, gsem, ssem,
             ones_v, deg_sh, dsem) = rest
        else:
            out_hbm, src_v, dst_v, buf_v, acc_sh, gsem, ssem = rest
        c = lax.axis_index("c")
        s = lax.axis_index("s")
        wid = s * NC + c
        r0 = s * ROWS_PER_SUB
        pltpu.sync_copy(zD_hbm.at[pl.ds(r0, ROWS_PER_SUB)],
                        acc_sh.at[pl.ds(r0, ROWS_PER_SUB)])
        if with_deg:
            pltpu.sync_copy(z16_hbm.at[pl.ds(r0, ROWS_PER_SUB)],
                            deg_sh.at[pl.ds(r0, ROWS_PER_SUB)])
            pltpu.sync_copy(ones_hbm, ones_v)
        pltpu.sync_copy(src_hbm.at[wid], src_v)
        pltpu.sync_copy(dst_hbm.at[wid], dst_v)
        plsc.subcore_barrier()

        def wait_scatter(j, b):
            pltpu.make_async_copy(buf_v.at[j],
                                  acc_sh.at[dst_v.at[b]], ssem.at[j]).wait()
            if with_deg:
                pltpu.make_async_copy(ones_v,
                                      deg_sh.at[dst_v.at[b]],
                                      dsem.at[j]).wait()

        for b0 in range(LOOKAHEAD):
            pltpu.async_copy(y_hbm.at[src_v.at[b0]], buf_v.at[b0],
                             gsem.at[b0])

        @pl.loop(0, NB // NSLOT)
        def _(t):
            for j in range(NSLOT):
                b = t * NSLOT + j
                pltpu.make_async_copy(y_hbm.at[src_v.at[b]], buf_v.at[j],
                                      gsem.at[j]).wait()
                pltpu.async_copy(buf_v.at[j], acc_sh.at[dst_v.at[b]],
                                 ssem.at[j], add=True)
                if with_deg:
                    pltpu.async_copy(ones_v, deg_sh.at[dst_v.at[b]],
                                     dsem.at[j], add=True)
                wait_scatter(j, b)
                jf = (j + LOOKAHEAD) % NSLOT
                f = b + LOOKAHEAD

                @pl.when(f < NB)
                def _():
                    pltpu.async_copy(y_hbm.at[src_v.at[f]], buf_v.at[jf],
                                     gsem.at[jf])

        plsc.subcore_barrier()
        pltpu.sync_copy(acc_sh.at[pl.ds(r0, ROWS_PER_SUB)],
                        out_hbm.at[c, pl.ds(r0, ROWS_PER_SUB)])
        if with_deg:
            pltpu.sync_copy(deg_sh.at[pl.ds(r0, ROWS_PER_SUB)],
                            deg_hbm.at[c, pl.ds(r0, ROWS_PER_SUB)])

    return k(y, src_r, dst_r, ones_blk, zD, z16)


def _dot(a, b):
    return lax.dot_general(a, b, (((1,), (0,)), ((), ())),
                           preferred_element_type=_F32,
                           precision=lax.Precision.HIGHEST)


def _tc_proj1(x, W_l, W_r, b):
    def body(x_ref, wl_ref, wr_ref, b_ref, y_ref, r_ref):
        xv = x_ref[...]
        y_ref[...] = _dot(xv, wl_ref[...])
        r_ref[...] = _dot(xv, wr_ref[...]) + b_ref[...]

    return pl.pallas_call(
        body,
        out_shape=(jax.ShapeDtypeStruct((N_NODES, D_H1), _F32),
                   jax.ShapeDtypeStruct((N_NODES, D_H1), _F32)),
    )(x, W_l, W_r, b)


def _tc_mid(p1, pdeg, r1, W2_l, W2_r, b2):
    def body(p1_ref, pdeg_ref, r1_ref, wl_ref, wr_ref, b_ref,
             y2_ref, r2_ref):
        deg = pdeg_ref[0, :N_NODES, 0:1] + pdeg_ref[1, :N_NODES, 0:1]
        degc = jnp.maximum(deg, 1.0)
        h = jnp.maximum(
            (p1_ref[0, :N_NODES, :] + p1_ref[1, :N_NODES, :]) / degc
            + r1_ref[...], 0.0)
        y2_ref[...] = _dot(h, wl_ref[...])
        r2_ref[...] = _dot(h, wr_ref[...]) + b_ref[...]

    return pl.pallas_call(
        body,
        out_shape=(jax.ShapeDtypeStruct((N_NODES, D_H2), _F32),
                   jax.ShapeDtypeStruct((N_NODES, D_H2), _F32)),
    )(p1, pdeg, r1, W2_l, W2_r, b2)


def _tc_final(p2, pdeg, r2):
    def body(p2_ref, pdeg_ref, r2_ref, o_ref):
        deg = pdeg_ref[0, :N_NODES, 0:1] + pdeg_ref[1, :N_NODES, 0:1]
        degc = jnp.maximum(deg, 1.0)
        o_ref[...] = ((p2_ref[0, :N_NODES, :] + p2_ref[1, :N_NODES, :])
                      / degc + r2_ref[...])

    return pl.pallas_call(
        body,
        out_shape=jax.ShapeDtypeStruct((N_NODES, D_H2), _F32),
    )(p2, pdeg, r2)


def kernel(x, edge_index, W1_l, W1_r, b1, W2_l, W2_r, b2):
    src_r = edge_index[0].reshape(NW, NB, BLK)
    dst_r = edge_index[1].reshape(NW, NB, BLK)
    ones_blk = jnp.ones((BLK, 16), _F32)
    z64 = jnp.zeros((N_PAD, D_H1), _F32)
    z32 = jnp.zeros((N_PAD, D_H2), _F32)
    z16 = jnp.zeros((N_PAD, 16), _F32)

    y1, r1 = _tc_proj1(x, W1_l, W1_r, b1.reshape(1, D_H1))
    p1, pdeg = _sc_segsum(y1, src_r, dst_r, ones_blk, z64, z16, True)
    y2, r2 = _tc_mid(p1, pdeg, r1, W2_l, W2_r, b2.reshape(1, D_H2))
    p2 = _sc_segsum(y2, src_r, dst_r, ones_blk, z32, z16, False)
    if isinstance(p2, (tuple, list)):
        p2 = p2[0]
    return _tc_final(p2, pdeg, r2)


# speedup vs baseline: 16.8309x; 1.0493x over previous
import functools

import jax
import jax.numpy as jnp
from jax import lax
from jax.experimental import pallas as pl
from jax.experimental.pallas import tpu as pltpu
from jax.experimental.pallas import tpu_sc as plsc

N_NODES = 10000
N_EDGES = 320000
D_IN = 128
D_H1 = 64
D_H2 = 32

NC = 2
NS = 16
NW = NC * NS
E_PER_W = N_EDGES // NW
BLK = 80
NB = E_PER_W // BLK
ROWS_PER_SUB = 632
N_PAD = ROWS_PER_SUB * NS
NSLOT = 5
LOOKAHEAD = 3

_MESH = plsc.VectorSubcoreMesh(core_axis_name="c", subcore_axis_name="s")

_F32 = jnp.float32


def _sc_segsum(y, src_r, dst_r, ones_blk, zD, z16, with_deg):
    d = y.shape[1]
    out_type = [jax.ShapeDtypeStruct((NC, N_PAD, d), _F32)]
    scratch = [
        pltpu.VMEM((NB, BLK), jnp.int32),
        pltpu.VMEM((NB, BLK), jnp.int32),
        pltpu.VMEM((NSLOT, BLK, d), _F32),
        pltpu.VMEM_SHARED((N_PAD, d), _F32),
        pltpu.SemaphoreType.DMA((NSLOT,)),
        pltpu.SemaphoreType.DMA((NSLOT,)),
    ]
    if with_deg:
        out_type.append(jax.ShapeDtypeStruct((NC, N_PAD, 16), _F32))
        scratch += [
            pltpu.VMEM((BLK, 16), _F32),
            pltpu.VMEM_SHARED((N_PAD, 16), _F32),
            pltpu.SemaphoreType.DMA((NSLOT,)),
        ]

    @functools.partial(
        pl.kernel,
        out_type=tuple(out_type),
        mesh=_MESH,
        scratch_types=tuple(scratch),
        compiler_params=pltpu.CompilerParams(use_tc_tiling_on_sc=False),
    )
    def k(y_hbm, src_hbm, dst_hbm, ones_hbm, zD_hbm, z16_hbm, *rest):
        if with_deg:
            (out_hbm, deg_hbm, src_v, dst_v, buf_v, acc_sh, gsem, ssem,
             ones_v, deg_sh, dsem) = rest
        else:
            out_hbm, src_v, dst_v, buf_v, acc_sh, gsem, ssem = rest
        c = lax.axis_index("c")
        s = lax.axis_index("s")
        wid = s * NC + c
        r0 = s * ROWS_PER_SUB
        pltpu.sync_copy(zD_hbm.at[pl.ds(r0, ROWS_PER_SUB)],
                        acc_sh.at[pl.ds(r0, ROWS_PER_SUB)])
        if with_deg:
            pltpu.sync_copy(z16_hbm.at[pl.ds(r0, ROWS_PER_SUB)],
                            deg_sh.at[pl.ds(r0, ROWS_PER_SUB)])
            pltpu.sync_copy(ones_hbm, ones_v)
        pltpu.sync_copy(src_hbm.at[wid], src_v)
        pltpu.sync_copy(dst_hbm.at[wid], dst_v)
        plsc.subcore_barrier()

        def wait_scatter(j, b):
            pltpu.make_async_copy(buf_v.at[j],
                                  acc_sh.at[dst_v.at[b]], ssem.at[j]).wait()
            if with_deg:
                pltpu.make_async_copy(ones_v,
                                      deg_sh.at[dst_v.at[b]],
                                      dsem.at[j]).wait()

        for b0 in range(LOOKAHEAD):
            pltpu.async_copy(y_hbm.at[src_v.at[b0]], buf_v.at[b0],
                             gsem.at[b0])

        @pl.loop(0, NB // NSLOT)
        def _(t):
            for j in range(NSLOT):
                b = t * NSLOT + j
                pltpu.make_async_copy(y_hbm.at[src_v.at[b]], buf_v.at[j],
                                      gsem.at[j]).wait()
                jp = (j - 1) % NSLOT
                if j == 0:
                    @pl.when(t > 0)
                    def _():
                        wait_scatter(jp, b)
                else:
                    wait_scatter(jp, b)
                pltpu.async_copy(buf_v.at[j], acc_sh.at[dst_v.at[b]],
                                 ssem.at[j], add=True)
                if with_deg:
                    pltpu.async_copy(ones_v, deg_sh.at[dst_v.at[b]],
                                     dsem.at[j], add=True)
                jf = (j + LOOKAHEAD) % NSLOT
                f = b + LOOKAHEAD

                @pl.when(f < NB)
                def _():
                    pltpu.async_copy(y_hbm.at[src_v.at[f]], buf_v.at[jf],
                                     gsem.at[jf])

        wait_scatter(NSLOT - 1, 0)
        plsc.subcore_barrier()
        pltpu.sync_copy(acc_sh.at[pl.ds(r0, ROWS_PER_SUB)],
                        out_hbm.at[c, pl.ds(r0, ROWS_PER_SUB)])
        if with_deg:
            pltpu.sync_copy(deg_sh.at[pl.ds(r0, ROWS_PER_SUB)],
                            deg_hbm.at[c, pl.ds(r0, ROWS_PER_SUB)])

    return k(y, src_r, dst_r, ones_blk, zD, z16)


def _dot(a, b):
    return lax.dot_general(a, b, (((1,), (0,)), ((), ())),
                           preferred_element_type=_F32,
                           precision=lax.Precision.HIGHEST)


def _tc_proj1(x, W_l, W_r, b):
    def body(x_ref, wl_ref, wr_ref, b_ref, y_ref, r_ref):
        xv = x_ref[...]
        y_ref[...] = _dot(xv, wl_ref[...])
        r_ref[...] = _dot(xv, wr_ref[...]) + b_ref[...]

    return pl.pallas_call(
        body,
        out_shape=(jax.ShapeDtypeStruct((N_NODES, D_H1), _F32),
                   jax.ShapeDtypeStruct((N_NODES, D_H1), _F32)),
    )(x, W_l, W_r, b)


def _tc_mid(p1, pdeg, r1, W2_l, W2_r, b2):
    def body(p1_ref, pdeg_ref, r1_ref, wl_ref, wr_ref, b_ref,
             y2_ref, r2_ref):
        deg = pdeg_ref[0, :N_NODES, 0:1] + pdeg_ref[1, :N_NODES, 0:1]
        degc = jnp.maximum(deg, 1.0)
        h = jnp.maximum(
            (p1_ref[0, :N_NODES, :] + p1_ref[1, :N_NODES, :]) / degc
            + r1_ref[...], 0.0)
        y2_ref[...] = _dot(h, wl_ref[...])
        r2_ref[...] = _dot(h, wr_ref[...]) + b_ref[...]

    return pl.pallas_call(
        body,
        out_shape=(jax.ShapeDtypeStruct((N_NODES, D_H2), _F32),
                   jax.ShapeDtypeStruct((N_NODES, D_H2), _F32)),
    )(p1, pdeg, r1, W2_l, W2_r, b2)


def _tc_final(p2, pdeg, r2):
    def body(p2_ref, pdeg_ref, r2_ref, o_ref):
        deg = pdeg_ref[0, :N_NODES, 0:1] + pdeg_ref[1, :N_NODES, 0:1]
        degc = jnp.maximum(deg, 1.0)
        o_ref[...] = ((p2_ref[0, :N_NODES, :] + p2_ref[1, :N_NODES, :])
                      / degc + r2_ref[...])

    return pl.pallas_call(
        body,
        out_shape=jax.ShapeDtypeStruct((N_NODES, D_H2), _F32),
    )(p2, pdeg, r2)


def kernel(x, edge_index, W1_l, W1_r, b1, W2_l, W2_r, b2):
    src_r = edge_index[0].reshape(NW, NB, BLK)
    dst_r = edge_index[1].reshape(NW, NB, BLK)
    ones_blk = jnp.ones((BLK, 16), _F32)
    z64 = jnp.zeros((N_PAD, D_H1), _F32)
    z32 = jnp.zeros((N_PAD, D_H2), _F32)
    z16 = jnp.zeros((N_PAD, 16), _F32)

    y1, r1 = _tc_proj1(x, W1_l, W1_r, b1.reshape(1, D_H1))
    p1, pdeg = _sc_segsum(y1, src_r, dst_r, ones_blk, z64, z16, True)
    y2, r2 = _tc_mid(p1, pdeg, r1, W2_l, W2_r, b2.reshape(1, D_H2))
    p2 = _sc_segsum(y2, src_r, dst_r, ones_blk, z32, z16, False)
    if isinstance(p2, (tuple, list)):
        p2 = p2[0]
    return _tc_final(p2, pdeg, r2)
